# Initial kernel scaffold; baseline (speedup 1.0000x reference)
import jax
import jax.numpy as jnp
from jax.experimental import pallas as pl


def kernel(x, fc1_w, fc1_b, fc2_w, fc2_b):
    raise NotImplementedError("write your pallas kernel here")


# speedup vs baseline: 1.0932x; 1.0932x over previous
import functools

import jax
import jax.numpy as jnp
from jax.experimental import pallas as pl
from jax.experimental.pallas import tpu as pltpu


def _se_body(x_ref, w1t_ref, b1_ref, w2_ref, b2_ref, o_ref, *, inv_spatial):
    xb = x_ref[0]

    s = jnp.sum(xb, axis=1, keepdims=True).astype(jnp.float32) * inv_spatial

    w1t = w1t_ref[...].astype(jnp.float32)
    h = jnp.sum(w1t * s, axis=0, keepdims=True)
    h = jnp.maximum(h + b1_ref[...].astype(jnp.float32), 0.0)
    w2 = w2_ref[...].astype(jnp.float32)
    g = jnp.sum(w2 * h, axis=1, keepdims=True)
    g = g + b2_ref[...].astype(jnp.float32)
    gate = (1.0 / (1.0 + jnp.exp(-g))).astype(xb.dtype)

    o_ref[0] = (xb * gate).astype(o_ref.dtype)


def kernel(x, fc1_w, fc1_b, fc2_w, fc2_b):
    N, C, D, H, W = x.shape
    Cr = fc1_w.shape[0]
    S = D * H * W
    S_pad = ((S + 127) // 128) * 128

    x2 = x.reshape(N, C, S)
    if S_pad != S:
        x2 = jnp.pad(x2, ((0, 0), (0, 0), (0, S_pad - S)))

    w1t = jnp.transpose(fc1_w)
    b1r = fc1_b.reshape(1, Cr)
    b2c = fc2_b.reshape(C, 1)

    out2 = pl.pallas_call(
        functools.partial(_se_body, inv_spatial=1.0 / S),
        out_shape=jax.ShapeDtypeStruct((N, C, S_pad), x.dtype),
        grid=(N,),
        in_specs=[
            pl.BlockSpec((1, C, S_pad), lambda n: (n, 0, 0)),
            pl.BlockSpec((C, Cr), lambda n: (0, 0)),
            pl.BlockSpec((1, Cr), lambda n: (0, 0)),
            pl.BlockSpec((C, Cr), lambda n: (0, 0)),
            pl.BlockSpec((C, 1), lambda n: (0, 0)),
        ],
        out_specs=pl.BlockSpec((1, C, S_pad), lambda n: (n, 0, 0)),
        compiler_params=pltpu.CompilerParams(
            dimension_semantics=("parallel",),
            vmem_limit_bytes=100 * 1024 * 1024),
    )(x2, w1t, b1r, fc2_w, b2c)

    if S_pad != S:
        out2 = out2[:, :, :S]
    return out2.reshape(N, C, D, H, W)


# speedup vs baseline: 3.9005x; 3.5679x over previous
import functools

import jax
import jax.numpy as jnp
from jax.experimental import pallas as pl
from jax.experimental.pallas import tpu as pltpu


def _se_body(x_ref, w1_ref, b1_ref, w2t_ref, b2_ref, o_ref, *, inv_spatial):
    xb = x_ref[0]

    s = jnp.sum(xb, axis=0, keepdims=True).astype(jnp.float32) * inv_spatial

    w1 = w1_ref[...].astype(jnp.float32)
    h = jnp.sum(w1 * s, axis=1, keepdims=True)
    h = jnp.maximum(h + b1_ref[...].astype(jnp.float32), 0.0)

    w2t = w2t_ref[...].astype(jnp.float32)
    g = jnp.sum(w2t * h, axis=0, keepdims=True)
    g = g + b2_ref[...].astype(jnp.float32)
    gate = (1.0 / (1.0 + jnp.exp(-g))).astype(xb.dtype)

    o_ref[0] = (xb * gate).astype(o_ref.dtype)


def kernel(x, fc1_w, fc1_b, fc2_w, fc2_b):
    N, C, D, H, W = x.shape
    Cr = fc1_w.shape[0]
    S = D * H * W

    xt = jnp.transpose(x, (0, 2, 3, 4, 1)).reshape(N, S, C)

    w2t = jnp.transpose(fc2_w)
    b1c = fc1_b.reshape(Cr, 1)
    b2r = fc2_b.reshape(1, C)

    out = pl.pallas_call(
        functools.partial(_se_body, inv_spatial=1.0 / S),
        out_shape=jax.ShapeDtypeStruct((N, S, C), x.dtype),
        grid=(N,),
        in_specs=[
            pl.BlockSpec((1, S, C), lambda n: (n, 0, 0)),
            pl.BlockSpec((Cr, C), lambda n: (0, 0)),
            pl.BlockSpec((Cr, 1), lambda n: (0, 0)),
            pl.BlockSpec((Cr, C), lambda n: (0, 0)),
            pl.BlockSpec((1, C), lambda n: (0, 0)),
        ],
        out_specs=pl.BlockSpec((1, S, C), lambda n: (n, 0, 0)),
        compiler_params=pltpu.CompilerParams(
            dimension_semantics=("parallel",),
            vmem_limit_bytes=100 * 1024 * 1024),
    )(xt, fc1_w, b1c, w2t, b2r)

    return out.reshape(N, D, H, W, C).transpose(0, 4, 1, 2, 3)


# speedup vs baseline: 4.1950x; 1.0755x over previous
import functools

import jax
import jax.numpy as jnp
from jax.experimental import pallas as pl
from jax.experimental.pallas import tpu as pltpu


def _se_body(x_ref, w1_ref, b1_ref, w2t_ref, b2_ref, o_ref, *, inv_spatial,
             batches_per_block):
    for i in range(batches_per_block):
        xb = x_ref[i]

        s = jnp.sum(xb, axis=0, keepdims=True).astype(jnp.float32) * inv_spatial

        w1 = w1_ref[...].astype(jnp.float32)
        h = jnp.sum(w1 * s, axis=1, keepdims=True)
        h = jnp.maximum(h + b1_ref[...].astype(jnp.float32), 0.0)

        w2t = w2t_ref[...].astype(jnp.float32)
        g = jnp.sum(w2t * h, axis=0, keepdims=True)
        g = g + b2_ref[...].astype(jnp.float32)
        gate = (1.0 / (1.0 + jnp.exp(-g))).astype(xb.dtype)

        o_ref[i] = (xb * gate).astype(o_ref.dtype)


def kernel(x, fc1_w, fc1_b, fc2_w, fc2_b):
    N, C, D, H, W = x.shape
    Cr = fc1_w.shape[0]
    S = D * H * W

    xt = jnp.transpose(x, (0, 2, 3, 4, 1)).reshape(N, S, C)

    w2t = jnp.transpose(fc2_w)
    b1c = fc1_b.reshape(Cr, 1)
    b2r = fc2_b.reshape(1, C)

    bpb = 2 if N % 2 == 0 else 1

    out = pl.pallas_call(
        functools.partial(_se_body, inv_spatial=1.0 / S,
                          batches_per_block=bpb),
        out_shape=jax.ShapeDtypeStruct((N, S, C), x.dtype),
        grid=(N // bpb,),
        in_specs=[
            pl.BlockSpec((bpb, S, C), lambda n: (n, 0, 0)),
            pl.BlockSpec((Cr, C), lambda n: (0, 0)),
            pl.BlockSpec((Cr, 1), lambda n: (0, 0)),
            pl.BlockSpec((Cr, C), lambda n: (0, 0)),
            pl.BlockSpec((1, C), lambda n: (0, 0)),
        ],
        out_specs=pl.BlockSpec((bpb, S, C), lambda n: (n, 0, 0)),
        compiler_params=pltpu.CompilerParams(
            dimension_semantics=("parallel",),
            vmem_limit_bytes=100 * 1024 * 1024),
    )(xt, fc1_w, b1c, w2t, b2r)

    return out.reshape(N, D, H, W, C).transpose(0, 4, 1, 2, 3)
